# baseline (speedup 1.0000x reference)
import functools

import jax
import jax.numpy as jnp
from jax import lax
from jax.experimental import pallas as pl
from jax.experimental.pallas import tpu as pltpu
from jax.experimental.pallas import tpu_sc as plsc

S = 1024
NC = 2
NS = 16
NW = NC * NS
L = 16
CH = 80


def _logits_body(x_ref, w1_ref, b1_ref, w2_ref, b2_ref, w_ref, gmax_ref):
    i = pl.program_id(0)
    h = jnp.tanh(
        jax.lax.dot(x_ref[...], w1_ref[...], preferred_element_type=jnp.float32)
        + b1_ref[...]
    )
    w = jnp.sum(h * w2_ref[...][None, :, 0], axis=1, keepdims=True) + b2_ref[...]
    w_ref[...] = w
    bmax = jnp.max(w, keepdims=True)

    @pl.when(i == 0)
    def _():
        gmax_ref[...] = bmax

    @pl.when(i > 0)
    def _():
        gmax_ref[...] = jnp.maximum(gmax_ref[...], bmax)


def _stage1(x, W1, b1, W2, b2, block):
    n, d = x.shape
    nblocks = n // block
    return pl.pallas_call(
        _logits_body,
        grid=(nblocks,),
        in_specs=[
            pl.BlockSpec((block, d), lambda i: (i, 0)),
            pl.BlockSpec((d, d), lambda i: (0, 0)),
            pl.BlockSpec((d,), lambda i: (0,)),
            pl.BlockSpec((d, 1), lambda i: (0, 0)),
            pl.BlockSpec((1, 1), lambda i: (0, 0)),
        ],
        out_specs=[
            pl.BlockSpec((block, 1), lambda i: (i, 0)),
            pl.BlockSpec((1, 1), lambda i: (0, 0)),
        ],
        out_shape=[
            jax.ShapeDtypeStruct((n, 1), jnp.float32),
            jax.ShapeDtypeStruct((1, 1), jnp.float32),
        ],
    )(x, W1, b1, W2, b2.reshape(1, 1))


def _sc_pool_body(rows_per_w, d, x_hbm, w_hbm, seg_hbm, gmax_hbm,
                  outn_hbm, outd_hbm,
                  xb, eb, wb, evec_b, segb, gv, zb, accn, accd):
    c = lax.axis_index("c")
    sid = lax.axis_index("s")
    wid = sid * NC + c
    base = wid * rows_per_w
    nchunk = rows_per_w // CH
    lane0 = (lax.broadcasted_iota(jnp.int32, (L,), 0) == 0).astype(jnp.float32)

    pltpu.sync_copy(gmax_hbm, gv)
    gvec = gv[...]

    rows_per_sid = S // NS

    def zrow(r, _):
        for g in range(d // L):
            zb[r, pl.ds(g * L, L)] = jnp.zeros((L,), jnp.float32)
        return 0

    lax.fori_loop(0, rows_per_sid, zrow, 0)

    def erow(r, _):
        for g in range(d // L):
            eb[r, pl.ds(g * L, L)] = jnp.zeros((L,), jnp.float32)
        return 0

    lax.fori_loop(0, CH, erow, 0)
    pltpu.sync_copy(zb, accn.at[pl.ds(sid * rows_per_sid, rows_per_sid)])
    pltpu.sync_copy(zb, accd.at[pl.ds(sid * rows_per_sid, rows_per_sid)])
    plsc.subcore_barrier()

    def chunk(k, _):
        r0 = base + k * CH
        pltpu.sync_copy(x_hbm.at[pl.ds(r0, CH)], xb)
        pltpu.sync_copy(w_hbm.at[pl.ds(r0, CH)], wb)
        pltpu.sync_copy(seg_hbm.at[pl.ds(r0, CH)], segb)
        for v in range(CH // L):
            evec_b[pl.ds(v * L, L)] = jnp.exp(wb[pl.ds(v * L, L)] - gvec)

        def row(r, _):
            er = plsc.load_gather(evec_b, [jnp.full((L,), r, jnp.int32)])
            for g in range(d // L):
                xb[r, pl.ds(g * L, L)] = xb[r, pl.ds(g * L, L)] * er
            eb[r, pl.ds(0, L)] = er * lane0
            return 0

        lax.fori_loop(0, CH, row, 0)

        pltpu.sync_copy(xb, accn.at[segb], add=True)
        pltpu.sync_copy(eb, accd.at[segb], add=True)
        return 0

    lax.fori_loop(0, nchunk, chunk, 0)
    plsc.subcore_barrier()
    sl = pl.ds(sid * rows_per_sid, rows_per_sid)
    pltpu.sync_copy(accn.at[sl], outn_hbm.at[c, sl])
    pltpu.sync_copy(accd.at[sl], outd_hbm.at[c, sl])


def _stage2_sc(x, w_flat, seg, gmax16):
    n, d = x.shape
    rows_per_w = n // NW
    mesh = plsc.VectorSubcoreMesh(
        core_axis_name="c", subcore_axis_name="s", num_cores=NC, num_subcores=NS
    )
    f = pl.kernel(
        functools.partial(_sc_pool_body, rows_per_w, d),
        out_type=[
            jax.ShapeDtypeStruct((NC, S, d), jnp.float32),
            jax.ShapeDtypeStruct((NC, S, d), jnp.float32),
        ],
        mesh=mesh,
        compiler_params=pltpu.CompilerParams(needs_layout_passes=False),
        scratch_types=[
            pltpu.VMEM((CH, d), jnp.float32),
            pltpu.VMEM((CH, d), jnp.float32),
            pltpu.VMEM((CH,), jnp.float32),
            pltpu.VMEM((CH,), jnp.float32),
            pltpu.VMEM((CH,), jnp.int32),
            pltpu.VMEM((L,), jnp.float32),
            pltpu.VMEM((S // NS, d), jnp.float32),
            pltpu.VMEM_SHARED((S, d), jnp.float32),
            pltpu.VMEM_SHARED((S, d), jnp.float32),
        ],
    )
    return f(x, w_flat, seg, gmax16)


def _combine_body(pn_ref, pd_ref, out_ref):
    numer = pn_ref[0] + pn_ref[1]
    denom = pd_ref[0, :, :1] + pd_ref[1, :, :1]
    out_ref[...] = jnp.where(denom > 0.0, numer / denom, 0.0)


def _stage3(pn, pd, d):
    return pl.pallas_call(
        _combine_body,
        in_specs=[
            pl.BlockSpec((NC, S, d), lambda: (0, 0, 0)),
            pl.BlockSpec((NC, S, d), lambda: (0, 0, 0)),
        ],
        out_specs=pl.BlockSpec((S, d), lambda: (0, 0)),
        out_shape=jax.ShapeDtypeStruct((S, d), jnp.float32),
    )(pn, pd)


def kernel(x, batch, W1, b1, W2, b2):
    n, d = x.shape
    seg = batch.astype(jnp.int32)
    w2d, gmax = _stage1(x, W1, b1, W2, b2, block=2000)
    w_flat = w2d.reshape(n)
    gmax16 = jnp.broadcast_to(gmax.reshape(1), (L,))
    pn, pd = _stage2_sc(x, w_flat, seg, gmax16)
    return _stage3(pn, pd, d)


# speedup vs baseline: 1.4085x; 1.4085x over previous
import functools

import jax
import jax.numpy as jnp
from jax import lax
from jax.experimental import pallas as pl
from jax.experimental.pallas import tpu as pltpu
from jax.experimental.pallas import tpu_sc as plsc

S = 1024
NC = 2
NS = 16
NW = NC * NS
L = 16
SUB = 80
GCH = 2000
NSUB = GCH // SUB
NBUF = 3


def _logits_body(x_ref, w1_ref, b1_ref, w2_ref, b2_ref, w_ref, gmax_ref):
    i = pl.program_id(0)
    h = jnp.tanh(
        jax.lax.dot(x_ref[...], w1_ref[...], preferred_element_type=jnp.float32)
        + b1_ref[...]
    )
    w = jnp.sum(h * w2_ref[...][None, :, 0], axis=1, keepdims=True) + b2_ref[...]
    w_ref[...] = w
    bmax = jnp.max(w, keepdims=True)

    @pl.when(i == 0)
    def _():
        gmax_ref[...] = bmax

    @pl.when(i > 0)
    def _():
        gmax_ref[...] = jnp.maximum(gmax_ref[...], bmax)


def _stage1(x, W1, b1, W2, b2, block):
    n, d = x.shape
    nblocks = n // block
    return pl.pallas_call(
        _logits_body,
        grid=(nblocks,),
        in_specs=[
            pl.BlockSpec((block, d), lambda i: (i, 0)),
            pl.BlockSpec((d, d), lambda i: (0, 0)),
            pl.BlockSpec((d,), lambda i: (0,)),
            pl.BlockSpec((d, 1), lambda i: (0, 0)),
            pl.BlockSpec((1, 1), lambda i: (0, 0)),
        ],
        out_specs=[
            pl.BlockSpec((block, 1), lambda i: (i, 0)),
            pl.BlockSpec((1, 1), lambda i: (0, 0)),
        ],
        out_shape=[
            jax.ShapeDtypeStruct((n, 1), jnp.float32),
            jax.ShapeDtypeStruct((1, 1), jnp.float32),
        ],
    )(x, W1, b1, W2, b2.reshape(1, 1))


def _sc_pool_body(rows_per_w, d, x_hbm, w_hbm, seg2d_hbm, gmax_hbm,
                  outn_hbm, outd_hbm,
                  xb, eb, wb, evec_b, segb0, segb1, segb2, gv, zb, accn, accd,
                  lsem, ssem, xsem, esem):
    sgb = (segb0, segb1, segb2)
    c = lax.axis_index("c")
    sid = lax.axis_index("s")
    wid = sid * NC + c
    base = wid * rows_per_w
    ngroups = rows_per_w // GCH
    lane0 = (lax.broadcasted_iota(jnp.int32, (L,), 0) == 0).astype(jnp.float32)

    pltpu.sync_copy(gmax_hbm, gv)
    gvec = gv[...]

    rows_per_sid = S // NS

    def zrow(r, _):
        for g in range(d // L):
            zb[r, pl.ds(g * L, L)] = jnp.zeros((L,), jnp.float32)
        return 0

    lax.fori_loop(0, rows_per_sid, zrow, 0)

    def erow(r, _):
        for b in range(NBUF):
            for g in range(d // L):
                eb[b, r, pl.ds(g * L, L)] = jnp.zeros((L,), jnp.float32)
        return 0

    lax.fori_loop(0, SUB, erow, 0)
    pltpu.sync_copy(zb, accn.at[pl.ds(sid * rows_per_sid, rows_per_sid)])
    pltpu.sync_copy(zb, accd.at[pl.ds(sid * rows_per_sid, rows_per_sid)])
    plsc.subcore_barrier()

    def group(g, _):
        gbase = base + g * GCH
        srow = wid * (rows_per_w // SUB) + g * NSUB
        pltpu.sync_copy(w_hbm.at[pl.ds(gbase, GCH)], wb)

        def ev(v, _):
            evec_b[pl.ds(v * L, L)] = jnp.exp(wb[pl.ds(v * L, L)] - gvec)
            return 0

        lax.fori_loop(0, GCH // L, ev, 0)

        pltpu.async_copy(x_hbm.at[pl.ds(gbase, SUB)], xb.at[0], lsem.at[0])
        pltpu.async_copy(seg2d_hbm.at[srow], sgb[0], ssem.at[0])
        for j in range(NSUB):
            s = j % NBUF
            pltpu.make_async_copy(x_hbm.at[pl.ds(gbase + j * SUB, SUB)],
                                  xb.at[s], lsem.at[s]).wait()
            pltpu.make_async_copy(seg2d_hbm.at[srow + j], sgb[s],
                                  ssem.at[s]).wait()
            if j + 1 < NSUB:
                ns = (j + 1) % NBUF
                if j >= 2:
                    pltpu.make_async_copy(xb.at[ns], accn.at[sgb[ns]],
                                          xsem.at[ns]).wait()
                    pltpu.make_async_copy(eb.at[ns], accd.at[sgb[ns]],
                                          esem.at[ns]).wait()
                pltpu.async_copy(x_hbm.at[pl.ds(gbase + (j + 1) * SUB, SUB)],
                                 xb.at[ns], lsem.at[ns])
                pltpu.async_copy(seg2d_hbm.at[srow + j + 1], sgb[ns],
                                 ssem.at[ns])

            xs = xb.at[s]
            es = eb.at[s]

            @plsc.parallel_loop(0, SUB, 1, unroll=4)
            def _(r):
                er = plsc.load_gather(
                    evec_b, [jnp.full((L,), j * SUB + r, jnp.int32)])
                for gg in range(d // L):
                    xs[r, pl.ds(gg * L, L)] = xs[r, pl.ds(gg * L, L)] * er
                es[r, pl.ds(0, L)] = er * lane0

            pltpu.async_copy(xs, accn.at[sgb[s]], xsem.at[s], add=True)
            pltpu.async_copy(es, accd.at[sgb[s]], esem.at[s], add=True)

        for jj in range(NSUB - 3, NSUB):
            ss = jj % NBUF
            pltpu.make_async_copy(xb.at[ss], accn.at[sgb[ss]],
                                  xsem.at[ss]).wait()
            pltpu.make_async_copy(eb.at[ss], accd.at[sgb[ss]],
                                  esem.at[ss]).wait()
        return 0

    lax.fori_loop(0, ngroups, group, 0)
    plsc.subcore_barrier()
    sl = pl.ds(sid * rows_per_sid, rows_per_sid)
    pltpu.sync_copy(accn.at[sl], outn_hbm.at[c, sl])
    pltpu.sync_copy(accd.at[sl], outd_hbm.at[c, sl])


def _stage2_sc(x, w_flat, seg2d, gmax16):
    n, d = x.shape
    rows_per_w = n // NW
    mesh = plsc.VectorSubcoreMesh(
        core_axis_name="c", subcore_axis_name="s", num_cores=NC, num_subcores=NS
    )
    f = pl.kernel(
        functools.partial(_sc_pool_body, rows_per_w, d),
        out_type=[
            jax.ShapeDtypeStruct((NC, S, d), jnp.float32),
            jax.ShapeDtypeStruct((NC, S, d), jnp.float32),
        ],
        mesh=mesh,
        compiler_params=pltpu.CompilerParams(needs_layout_passes=False),
        scratch_types=[
            pltpu.VMEM((NBUF, SUB, d), jnp.float32),
            pltpu.VMEM((NBUF, SUB, d), jnp.float32),
            pltpu.VMEM((GCH,), jnp.float32),
            pltpu.VMEM((GCH,), jnp.float32),
            pltpu.VMEM((SUB,), jnp.int32),
            pltpu.VMEM((SUB,), jnp.int32),
            pltpu.VMEM((SUB,), jnp.int32),
            pltpu.VMEM((L,), jnp.float32),
            pltpu.VMEM((S // NS, d), jnp.float32),
            pltpu.VMEM_SHARED((S, d), jnp.float32),
            pltpu.VMEM_SHARED((S, d), jnp.float32),
            pltpu.SemaphoreType.DMA((NBUF,)),
            pltpu.SemaphoreType.DMA((NBUF,)),
            pltpu.SemaphoreType.DMA((NBUF,)),
            pltpu.SemaphoreType.DMA((NBUF,)),
        ],
    )
    return f(x, w_flat, seg2d, gmax16)


def _combine_body(pn_ref, pd_ref, out_ref):
    numer = pn_ref[0] + pn_ref[1]
    denom = pd_ref[0, :, :1] + pd_ref[1, :, :1]
    out_ref[...] = jnp.where(denom > 0.0, numer / denom, 0.0)


def _stage3(pn, pd, d):
    return pl.pallas_call(
        _combine_body,
        in_specs=[
            pl.BlockSpec((NC, S, d), lambda: (0, 0, 0)),
            pl.BlockSpec((NC, S, d), lambda: (0, 0, 0)),
        ],
        out_specs=pl.BlockSpec((S, d), lambda: (0, 0)),
        out_shape=jax.ShapeDtypeStruct((S, d), jnp.float32),
    )(pn, pd)


def kernel(x, batch, W1, b1, W2, b2):
    n, d = x.shape
    seg = batch.astype(jnp.int32)
    w2d, gmax = _stage1(x, W1, b1, W2, b2, block=2000)
    w_flat = w2d.reshape(n)
    seg2d = seg.reshape(n // SUB, SUB)
    gmax16 = jnp.broadcast_to(gmax.reshape(1), (L,))
    pn, pd = _stage2_sc(x, w_flat, seg2d, gmax16)
    return _stage3(pn, pd, d)


# speedup vs baseline: 1.6752x; 1.1894x over previous
import functools

import jax
import jax.numpy as jnp
from jax import lax
from jax.experimental import pallas as pl
from jax.experimental.pallas import tpu as pltpu
from jax.experimental.pallas import tpu_sc as plsc

S = 1024
NC = 2
NS = 16
NW = NC * NS
L = 16
SUB = 80
GCH = 2000
NSUB = GCH // SUB
NBUF = 3


def _logits_body(x_ref, w1_ref, b1_ref, w2_ref, b2_ref, w_ref, gmax_ref):
    i = pl.program_id(0)
    h = jnp.tanh(
        jax.lax.dot(x_ref[...], w1_ref[...], preferred_element_type=jnp.float32)
        + b1_ref[...]
    )
    w = jax.lax.dot(h, w2_ref[...], preferred_element_type=jnp.float32) + b2_ref[...]
    w_ref[...] = w
    bmax = jnp.max(w, keepdims=True)

    @pl.when(i == 0)
    def _():
        gmax_ref[...] = bmax

    @pl.when(i > 0)
    def _():
        gmax_ref[...] = jnp.maximum(gmax_ref[...], bmax)


def _stage1(x, W1, b1, W2, b2, block):
    n, d = x.shape
    nblocks = n // block
    return pl.pallas_call(
        _logits_body,
        grid=(nblocks,),
        in_specs=[
            pl.BlockSpec((block, d), lambda i: (i, 0)),
            pl.BlockSpec((d, d), lambda i: (0, 0)),
            pl.BlockSpec((d,), lambda i: (0,)),
            pl.BlockSpec((d, 1), lambda i: (0, 0)),
            pl.BlockSpec((1, 1), lambda i: (0, 0)),
        ],
        out_specs=[
            pl.BlockSpec((block, 1), lambda i: (i, 0)),
            pl.BlockSpec((1, 1), lambda i: (0, 0)),
        ],
        out_shape=[
            jax.ShapeDtypeStruct((n, 1), jnp.float32),
            jax.ShapeDtypeStruct((1, 1), jnp.float32),
        ],
    )(x, W1, b1, W2, b2.reshape(1, 1))


def _sc_pool_body(rows_per_w, d, x_hbm, w_hbm, seg2d_hbm, gmax_hbm,
                  outn_hbm, outd_hbm,
                  xb, eb, wb, evec_b, segb0, segb1, segb2, gv, zb, accn, accd,
                  lsem, ssem, xsem, esem):
    sgb = (segb0, segb1, segb2)
    c = lax.axis_index("c")
    sid = lax.axis_index("s")
    wid = sid * NC + c
    base = wid * rows_per_w
    ngroups = rows_per_w // GCH
    lane0 = (lax.broadcasted_iota(jnp.int32, (L,), 0) == 0).astype(jnp.float32)

    pltpu.sync_copy(gmax_hbm, gv)
    gvec = gv[...]

    rows_per_sid = S // NS

    def zrow(r, _):
        for g in range(d // L):
            zb[r, pl.ds(g * L, L)] = jnp.zeros((L,), jnp.float32)
        return 0

    lax.fori_loop(0, rows_per_sid, zrow, 0)

    def erow(r, _):
        for b in range(NBUF):
            for g in range(d // L):
                eb[b, r, pl.ds(g * L, L)] = jnp.zeros((L,), jnp.float32)
        return 0

    lax.fori_loop(0, SUB, erow, 0)
    pltpu.sync_copy(zb, accn.at[pl.ds(sid * rows_per_sid, rows_per_sid)])
    pltpu.sync_copy(zb, accd.at[pl.ds(sid * rows_per_sid, rows_per_sid)])
    plsc.subcore_barrier()

    def group(g, _):
        gbase = base + g * GCH
        srow = wid * (rows_per_w // SUB) + g * NSUB
        pltpu.sync_copy(w_hbm.at[pl.ds(gbase, GCH)], wb)

        def ev(v, _):
            evec_b[pl.ds(v * L, L)] = jnp.exp(wb[pl.ds(v * L, L)] - gvec)
            return 0

        lax.fori_loop(0, GCH // L, ev, 0)

        pltpu.async_copy(x_hbm.at[pl.ds(gbase, SUB)], xb.at[0], lsem.at[0])
        pltpu.async_copy(seg2d_hbm.at[srow], sgb[0], ssem.at[0])
        for j in range(NSUB):
            s = j % NBUF
            pltpu.make_async_copy(x_hbm.at[pl.ds(gbase + j * SUB, SUB)],
                                  xb.at[s], lsem.at[s]).wait()
            pltpu.make_async_copy(seg2d_hbm.at[srow + j], sgb[s],
                                  ssem.at[s]).wait()
            if j + 1 < NSUB:
                ns = (j + 1) % NBUF
                if j >= 2:
                    pltpu.make_async_copy(xb.at[ns], accn.at[sgb[ns]],
                                          xsem.at[ns]).wait()
                    pltpu.make_async_copy(eb.at[ns], accd.at[sgb[ns]],
                                          esem.at[ns]).wait()
                pltpu.async_copy(x_hbm.at[pl.ds(gbase + (j + 1) * SUB, SUB)],
                                 xb.at[ns], lsem.at[ns])
                pltpu.async_copy(seg2d_hbm.at[srow + j + 1], sgb[ns],
                                 ssem.at[ns])

            xs = xb.at[s]
            es = eb.at[s]

            @plsc.parallel_loop(0, SUB, 1, unroll=4)
            def _(r):
                er = plsc.load_gather(
                    evec_b, [jnp.full((L,), j * SUB + r, jnp.int32)])
                for gg in range(d // L):
                    xs[r, pl.ds(gg * L, L)] = xs[r, pl.ds(gg * L, L)] * er
                es[r, pl.ds(0, L)] = er * lane0

            pltpu.async_copy(xs, accn.at[sgb[s]], xsem.at[s], add=True)
            pltpu.async_copy(es, accd.at[sgb[s]], esem.at[s], add=True)

        for jj in range(NSUB - 3, NSUB):
            ss = jj % NBUF
            pltpu.make_async_copy(xb.at[ss], accn.at[sgb[ss]],
                                  xsem.at[ss]).wait()
            pltpu.make_async_copy(eb.at[ss], accd.at[sgb[ss]],
                                  esem.at[ss]).wait()
        return 0

    lax.fori_loop(0, ngroups, group, 0)
    plsc.subcore_barrier()
    sl = pl.ds(sid * rows_per_sid, rows_per_sid)
    pltpu.sync_copy(accn.at[sl], outn_hbm.at[c, sl])
    pltpu.sync_copy(accd.at[sl], outd_hbm.at[c, sl])


def _stage2_sc(x, w_flat, seg2d, gmax16):
    n, d = x.shape
    rows_per_w = n // NW
    mesh = plsc.VectorSubcoreMesh(
        core_axis_name="c", subcore_axis_name="s", num_cores=NC, num_subcores=NS
    )
    f = pl.kernel(
        functools.partial(_sc_pool_body, rows_per_w, d),
        out_type=[
            jax.ShapeDtypeStruct((NC, S, d), jnp.float32),
            jax.ShapeDtypeStruct((NC, S, d), jnp.float32),
        ],
        mesh=mesh,
        compiler_params=pltpu.CompilerParams(needs_layout_passes=False),
        scratch_types=[
            pltpu.VMEM((NBUF, SUB, d), jnp.float32),
            pltpu.VMEM((NBUF, SUB, d), jnp.float32),
            pltpu.VMEM((GCH,), jnp.float32),
            pltpu.VMEM((GCH,), jnp.float32),
            pltpu.VMEM((SUB,), jnp.int32),
            pltpu.VMEM((SUB,), jnp.int32),
            pltpu.VMEM((SUB,), jnp.int32),
            pltpu.VMEM((L,), jnp.float32),
            pltpu.VMEM((S // NS, d), jnp.float32),
            pltpu.VMEM_SHARED((S, d), jnp.float32),
            pltpu.VMEM_SHARED((S, d), jnp.float32),
            pltpu.SemaphoreType.DMA((NBUF,)),
            pltpu.SemaphoreType.DMA((NBUF,)),
            pltpu.SemaphoreType.DMA((NBUF,)),
            pltpu.SemaphoreType.DMA((NBUF,)),
        ],
    )
    return f(x, w_flat, seg2d, gmax16)


def _combine_body(pn_ref, pd_ref, out_ref):
    numer = pn_ref[0] + pn_ref[1]
    denom = pd_ref[0, :, :1] + pd_ref[1, :, :1]
    out_ref[...] = jnp.where(denom > 0.0, numer / denom, 0.0)


def _stage3(pn, pd, d):
    return pl.pallas_call(
        _combine_body,
        in_specs=[
            pl.BlockSpec((NC, S, d), lambda: (0, 0, 0)),
            pl.BlockSpec((NC, S, d), lambda: (0, 0, 0)),
        ],
        out_specs=pl.BlockSpec((S, d), lambda: (0, 0)),
        out_shape=jax.ShapeDtypeStruct((S, d), jnp.float32),
    )(pn, pd)


def kernel(x, batch, W1, b1, W2, b2):
    n, d = x.shape
    seg = batch.astype(jnp.int32)
    w2d, gmax = _stage1(x, W1, b1, W2, b2, block=8000)
    w_flat = w2d.reshape(n)
    seg2d = seg.reshape(n // SUB, SUB)
    gmax16 = jnp.broadcast_to(gmax.reshape(1), (L,))
    pn, pd = _stage2_sc(x, w_flat, seg2d, gmax16)
    return _stage3(pn, pd, d)


# speedup vs baseline: 2.0327x; 1.2134x over previous
import functools

import jax
import jax.numpy as jnp
from jax import lax
from jax.experimental import pallas as pl
from jax.experimental.pallas import tpu as pltpu
from jax.experimental.pallas import tpu_sc as plsc

S = 1024
NC = 2
NS = 16
NW = NC * NS
L = 16
SUB = 80
GCH = 2000
NSUB = GCH // SUB
NBUF = 3


def _logits_body(x_ref, w1_ref, b1_ref, w2_ref, b2_ref, w_ref, gmax_ref):
    i = pl.program_id(0)
    h = jnp.tanh(
        jax.lax.dot(x_ref[...], w1_ref[...], preferred_element_type=jnp.float32)
        + b1_ref[...]
    )
    w = jax.lax.dot(h, w2_ref[...], preferred_element_type=jnp.float32) + b2_ref[...]
    w_ref[...] = w
    bmax = jnp.max(w, keepdims=True)

    @pl.when(i == 0)
    def _():
        gmax_ref[...] = bmax

    @pl.when(i > 0)
    def _():
        gmax_ref[...] = jnp.maximum(gmax_ref[...], bmax)


def _stage1(x, W1, b1, W2, b2, block, part_rows, part_base):
    n, d = x.shape
    nblocks = part_rows // block
    off = part_base // block
    return pl.pallas_call(
        _logits_body,
        grid=(nblocks,),
        in_specs=[
            pl.BlockSpec((block, d), lambda i: (i + off, 0)),
            pl.BlockSpec((d, d), lambda i: (0, 0)),
            pl.BlockSpec((d,), lambda i: (0,)),
            pl.BlockSpec((d, 1), lambda i: (0, 0)),
            pl.BlockSpec((1, 1), lambda i: (0, 0)),
        ],
        out_specs=[
            pl.BlockSpec((block, 1), lambda i: (i, 0)),
            pl.BlockSpec((1, 1), lambda i: (0, 0)),
        ],
        out_shape=[
            jax.ShapeDtypeStruct((part_rows, 1), jnp.float32),
            jax.ShapeDtypeStruct((1, 1), jnp.float32),
        ],
    )(x, W1, b1, W2, b2.reshape(1, 1))


def _sc_pool_body(rows_per_w, part_base, d, x_hbm, w_hbm, seg2d_hbm, gmax_hbm,
                  outn_hbm, outd_hbm,
                  xb, eb, wb, evec_b, segb0, segb1, segb2, gv, zb, accn, accd,
                  lsem, ssem, xsem, esem):
    sgb = (segb0, segb1, segb2)
    c = lax.axis_index("c")
    sid = lax.axis_index("s")
    wid = sid * NC + c
    base = part_base + wid * rows_per_w
    ngroups = rows_per_w // GCH
    lane0 = (lax.broadcasted_iota(jnp.int32, (L,), 0) == 0).astype(jnp.float32)

    pltpu.sync_copy(gmax_hbm, gv)
    gvec = gv[...]

    rows_per_sid = S // NS

    def zrow(r, _):
        for g in range(d // L):
            zb[r, pl.ds(g * L, L)] = jnp.zeros((L,), jnp.float32)
        return 0

    lax.fori_loop(0, rows_per_sid, zrow, 0)

    def erow(r, _):
        for b in range(NBUF):
            for g in range(d // L):
                eb[b, r, pl.ds(g * L, L)] = jnp.zeros((L,), jnp.float32)
        return 0

    lax.fori_loop(0, SUB, erow, 0)
    pltpu.sync_copy(zb, accn.at[pl.ds(sid * rows_per_sid, rows_per_sid)])
    pltpu.sync_copy(zb, accd.at[pl.ds(sid * rows_per_sid, rows_per_sid)])
    plsc.subcore_barrier()

    def group(g, _):
        gbase = base + g * GCH
        srow = (part_base // SUB) + wid * (rows_per_w // SUB) + g * NSUB
        pltpu.sync_copy(w_hbm.at[pl.ds(gbase, GCH)], wb)

        def ev(v, _):
            evec_b[pl.ds(v * L, L)] = jnp.exp(wb[pl.ds(v * L, L)] - gvec)
            return 0

        lax.fori_loop(0, GCH // L, ev, 0)

        pltpu.async_copy(x_hbm.at[pl.ds(gbase, SUB)], xb.at[0], lsem.at[0])
        pltpu.async_copy(seg2d_hbm.at[srow], sgb[0], ssem.at[0])
        for j in range(NSUB):
            s = j % NBUF
            pltpu.make_async_copy(x_hbm.at[pl.ds(gbase + j * SUB, SUB)],
                                  xb.at[s], lsem.at[s]).wait()
            pltpu.make_async_copy(seg2d_hbm.at[srow + j], sgb[s],
                                  ssem.at[s]).wait()
            if j + 1 < NSUB:
                ns = (j + 1) % NBUF
                if j >= 2:
                    pltpu.make_async_copy(xb.at[ns], accn.at[sgb[ns]],
                                          xsem.at[ns]).wait()
                    pltpu.make_async_copy(eb.at[ns], accd.at[sgb[ns]],
                                          esem.at[ns]).wait()
                pltpu.async_copy(x_hbm.at[pl.ds(gbase + (j + 1) * SUB, SUB)],
                                 xb.at[ns], lsem.at[ns])
                pltpu.async_copy(seg2d_hbm.at[srow + j + 1], sgb[ns],
                                 ssem.at[ns])

            xs = xb.at[s]
            es = eb.at[s]

            @plsc.parallel_loop(0, SUB, 1, unroll=4)
            def _(r):
                er = plsc.load_gather(
                    evec_b, [jnp.full((L,), j * SUB + r, jnp.int32)])
                for gg in range(d // L):
                    xs[r, pl.ds(gg * L, L)] = xs[r, pl.ds(gg * L, L)] * er
                es[r, pl.ds(0, L)] = er * lane0

            pltpu.async_copy(xs, accn.at[sgb[s]], xsem.at[s], add=True)
            pltpu.async_copy(es, accd.at[sgb[s]], esem.at[s], add=True)

        for jj in range(NSUB - 3, NSUB):
            ss = jj % NBUF
            pltpu.make_async_copy(xb.at[ss], accn.at[sgb[ss]],
                                  xsem.at[ss]).wait()
            pltpu.make_async_copy(eb.at[ss], accd.at[sgb[ss]],
                                  esem.at[ss]).wait()
        return 0

    lax.fori_loop(0, ngroups, group, 0)
    plsc.subcore_barrier()
    sl = pl.ds(sid * rows_per_sid, rows_per_sid)
    pltpu.sync_copy(accn.at[sl], outn_hbm.at[c, sl])
    pltpu.sync_copy(accd.at[sl], outd_hbm.at[c, sl])


def _stage2_sc(x, w_flat, seg2d, gmax16, part_rows, part_base):
    n, d = x.shape
    rows_per_w = part_rows // NW
    mesh = plsc.VectorSubcoreMesh(
        core_axis_name="c", subcore_axis_name="s", num_cores=NC, num_subcores=NS
    )
    f = pl.kernel(
        functools.partial(_sc_pool_body, rows_per_w, part_base, d),
        out_type=[
            jax.ShapeDtypeStruct((NC, S, d), jnp.float32),
            jax.ShapeDtypeStruct((NC, S, d), jnp.float32),
        ],
        mesh=mesh,
        compiler_params=pltpu.CompilerParams(needs_layout_passes=False),
        scratch_types=[
            pltpu.VMEM((NBUF, SUB, d), jnp.float32),
            pltpu.VMEM((NBUF, SUB, d), jnp.float32),
            pltpu.VMEM((GCH,), jnp.float32),
            pltpu.VMEM((GCH,), jnp.float32),
            pltpu.VMEM((SUB,), jnp.int32),
            pltpu.VMEM((SUB,), jnp.int32),
            pltpu.VMEM((SUB,), jnp.int32),
            pltpu.VMEM((L,), jnp.float32),
            pltpu.VMEM((S // NS, d), jnp.float32),
            pltpu.VMEM_SHARED((S, d), jnp.float32),
            pltpu.VMEM_SHARED((S, d), jnp.float32),
            pltpu.SemaphoreType.DMA((NBUF,)),
            pltpu.SemaphoreType.DMA((NBUF,)),
            pltpu.SemaphoreType.DMA((NBUF,)),
            pltpu.SemaphoreType.DMA((NBUF,)),
        ],
    )
    return f(x, w_flat, seg2d, gmax16)


def _combine_body(nparts, *refs):
    gmax_refs = refs[:nparts]
    pn_refs = refs[nparts:2 * nparts]
    pd_refs = refs[2 * nparts:3 * nparts]
    out_ref = refs[3 * nparts]
    m = gmax_refs[0][...]
    for p in range(1, nparts):
        m = jnp.maximum(m, gmax_refs[p][...])
    numer = jnp.zeros_like(out_ref)
    denom = jnp.zeros((S, 1), jnp.float32)
    for p in range(nparts):
        sc = jnp.exp(gmax_refs[p][...] - m)
        numer = numer + sc * (pn_refs[p][0] + pn_refs[p][1])
        denom = denom + sc * (pd_refs[p][0, :, :1] + pd_refs[p][1, :, :1])
    out_ref[...] = jnp.where(denom > 0.0, numer / denom, 0.0)


def _stage3(gmaxes, pns, pds, d):
    nparts = len(pns)
    gspec = [pl.BlockSpec((1, 1), lambda: (0, 0)) for _ in range(nparts)]
    tspec = [pl.BlockSpec((NC, S, d), lambda: (0, 0, 0))
             for _ in range(2 * nparts)]
    return pl.pallas_call(
        functools.partial(_combine_body, nparts),
        in_specs=gspec + tspec,
        out_specs=pl.BlockSpec((S, d), lambda: (0, 0)),
        out_shape=jax.ShapeDtypeStruct((S, d), jnp.float32),
    )(*gmaxes, *pns, *pds)


NPART = 5


def kernel(x, batch, W1, b1, W2, b2):
    n, d = x.shape
    seg = batch.astype(jnp.int32)
    seg2d = seg.reshape(n // SUB, SUB)
    part_rows = n // NPART
    gmaxes, pns, pds = [], [], []
    for p in range(NPART):
        w2d, gmax = _stage1(x, W1, b1, W2, b2, block=8000,
                            part_rows=part_rows, part_base=p * part_rows)
        gmax16 = jnp.broadcast_to(gmax.reshape(1), (L,))
        pn, pd = _stage2_sc(x, w2d.reshape(part_rows), seg2d, gmax16,
                            part_rows, p * part_rows)
        gmaxes.append(gmax)
        pns.append(pn)
        pds.append(pd)
    return _stage3(gmaxes, pns, pds, d)
